# baseline (speedup 1.0000x reference)
import functools
import jax
import jax.numpy as jnp
from jax import lax
from jax.experimental import pallas as pl
from jax.experimental.pallas import tpu as pltpu
from jax.experimental.pallas import tpu_sc as plsc

LANES = 16
N_CORES = 2
N_SUBCORES = 16
GRP = 8
TCW = 128
HF = 16


@jax.jit
def _run(user_ids, movie_ids, uf_t, mf_t):
    B = user_ids.shape[0]
    F = uf_t.shape[0]
    V = uf_t.shape[1]
    NW = N_CORES * N_SUBCORES
    BPW = B // NW
    NPAIR = BPW // (2 * GRP)

    n_full = (V // TCW) * TCW
    last_base = n_full - TCW
    tail_w = V - n_full

    mesh = plsc.VectorSubcoreMesh(core_axis_name="c", subcore_axis_name="s")

    @functools.partial(
        pl.kernel,
        mesh=mesh,
        compiler_params=pltpu.CompilerParams(needs_layout_passes=False),
        out_type=jax.ShapeDtypeStruct((B,), jnp.float32),
        scratch_types=[
            pltpu.VMEM((BPW,), jnp.int32),
            pltpu.VMEM((BPW,), jnp.int32),
            pltpu.VMEM((2, GRP, HF, TCW), jnp.float32),
            pltpu.VMEM((2, GRP, HF, TCW), jnp.float32),
            pltpu.VMEM((F, tail_w), jnp.float32),
            pltpu.VMEM((F, tail_w), jnp.float32),
            pltpu.VMEM((BPW,), jnp.float32),
            pltpu.SemaphoreType.DMA,
            pltpu.SemaphoreType.DMA,
        ],
    )
    def sc_kernel(uids_hbm, mids_hbm, uf_hbm, mf_hbm, out_hbm,
                  uidx_v, midx_v, uring_v, mring_v, utail_v, mtail_v,
                  out_v, sem_u, sem_m):
        wid = lax.axis_index("s") * N_CORES + lax.axis_index("c")
        base = wid * BPW

        pltpu.sync_copy(uids_hbm.at[pl.ds(base, BPW)], uidx_v)
        pltpu.sync_copy(mids_hbm.at[pl.ds(base, BPW)], midx_v)
        pltpu.sync_copy(uf_hbm.at[:, pl.ds(n_full, tail_w)], utail_v)
        pltpu.sync_copy(mf_hbm.at[:, pl.ds(n_full, tail_w)], mtail_v)

        lane = lax.broadcasted_iota(jnp.int32, (LANES,), 0)
        slot8 = jnp.bitwise_and(lane, GRP - 1)

        def bases_of(p):
            uvec = uidx_v[pl.ds(p * 2 * GRP, LANES)]
            mvec = midx_v[pl.ds(p * 2 * GRP, LANES)]
            ubase = jnp.minimum(jnp.bitwise_and(uvec, ~(TCW - 1)), last_base)
            mbase = jnp.minimum(jnp.bitwise_and(mvec, ~(TCW - 1)), last_base)
            return uvec, mvec, ubase, mbase

        def fire(s, lo, ubase, mbase):
            for j in range(GRP):
                pltpu.async_copy(
                    uf_hbm.at[pl.ds(s * HF, HF),
                              pl.ds(pl.multiple_of(ubase[lo + j], TCW), TCW)],
                    uring_v.at[s, j], sem_u)
                pltpu.async_copy(
                    mf_hbm.at[pl.ds(s * HF, HF),
                              pl.ds(pl.multiple_of(mbase[lo + j], TCW), TCW)],
                    mring_v.at[s, j], sem_m)

        def drain(s):
            for j in range(GRP):
                pltpu.make_async_copy(
                    uf_hbm.at[pl.ds(0, HF), pl.ds(0, TCW)],
                    uring_v.at[s, j], sem_u).wait()
                pltpu.make_async_copy(
                    mf_hbm.at[pl.ds(0, HF), pl.ds(0, TCW)],
                    mring_v.at[s, j], sem_m).wait()

        uvec0, mvec0, ubase0, mbase0 = bases_of(0)
        fire(0, 0, ubase0, mbase0)
        fire(1, 0, ubase0, mbase0)

        def pair(p, carry):
            uvec, mvec, ubase, mbase = bases_of(p)
            pn = jnp.minimum(p + 1, NPAIR - 1)
            uvecN, mvecN, ubaseN, mbaseN = bases_of(pn)

            uc = uvec - ubase
            mc = mvec - mbase
            ucl = jnp.minimum(uc, TCW - 1)
            mcl = jnp.minimum(mc, TCW - 1)
            uct = jnp.bitwise_and(uvec - n_full, tail_w - 1)
            mct = jnp.bitwise_and(mvec - n_full, tail_w - 1)
            u_is_tail = uvec >= n_full
            m_is_tail = mvec >= n_full

            def halfdot(s, ucol, mcol):
                acc = jnp.zeros((LANES,), jnp.float32)
                for fl in range(HF):
                    f = s * HF + fl
                    flv = jnp.full((LANES,), fl, jnp.int32)
                    fv = jnp.full((LANES,), f, jnp.int32)
                    sv = jnp.full((LANES,), s, jnp.int32)
                    u = plsc.load_gather(uring_v, [sv, slot8, flv, ucol])
                    m = plsc.load_gather(mring_v, [sv, slot8, flv, mcol])
                    ut = plsc.load_gather(utail_v, [fv, uct])
                    mt = plsc.load_gather(mtail_v, [fv, mct])
                    uv = jnp.where(u_is_tail, ut, u)
                    mv = jnp.where(m_is_tail, mt, m)
                    acc = acc + uv * mv
                return acc

            drain(0)
            drain(1)
            accA = halfdot(0, ucl, mcl) + halfdot(1, ucl, mcl)
            fire(0, GRP, ubase, mbase)
            fire(1, GRP, ubase, mbase)
            drain(0)
            drain(1)
            accB = halfdot(0, ucl, mcl) + halfdot(1, ucl, mcl)
            fire(0, 0, ubaseN, mbaseN)
            fire(1, 0, ubaseN, mbaseN)

            res = jnp.where(lane < GRP, accA, accB)
            out_v[pl.ds(p * 2 * GRP, LANES)] = res
            return carry

        lax.fori_loop(0, NPAIR, pair, 0)
        drain(0)
        drain(1)

        pltpu.sync_copy(out_v, out_hbm.at[pl.ds(base, BPW)])

    return sc_kernel(user_ids, movie_ids, uf_t, mf_t)


def kernel(user_ids, movie_ids, user_factors, movie_factors):
    out = _run(user_ids.astype(jnp.int32), movie_ids.astype(jnp.int32),
               user_factors.T, movie_factors.T)
    return out.reshape(-1, 1)
